# baseline (speedup 1.0000x reference)
import functools

import jax
import jax.numpy as jnp
from jax import lax
from jax.experimental import pallas as pl
from jax.experimental.pallas import tpu as pltpu
from jax.experimental.pallas import tpu_sc as plsc

_ALPHA = 4


def _bulk_copy_body(src, dst, sem):
    pltpu.make_async_copy(src, dst, sem).start()
    pltpu.make_async_copy(src, dst, sem).wait()


def _fast_copy(frames):
    return pl.pallas_call(
        _bulk_copy_body,
        out_shape=jax.ShapeDtypeStruct(frames.shape, frames.dtype),
        in_specs=[pl.BlockSpec(memory_space=pl.ANY)],
        out_specs=pl.BlockSpec(memory_space=pl.ANY),
        scratch_shapes=[pltpu.SemaphoreType.DMA],
    )(frames)


def kernel(frames):
    C, T, H, W = frames.shape
    TS = T // _ALPHA
    NSLICES = C * TS
    CHUNKS = 4
    RB = H // CHUNKS

    info = plsc.get_sparse_core_info()
    NC, NS = info.num_cores, info.num_subcores
    NW = NC * NS
    NTASK = NSLICES * CHUNKS
    TPW = NTASK // NW

    mesh = plsc.VectorSubcoreMesh(core_axis_name="c", subcore_axis_name="s")

    @functools.partial(
        pl.kernel,
        mesh=mesh,
        out_type=jax.ShapeDtypeStruct((C, TS, H, W), jnp.float32),
        scratch_types=[
            pltpu.VMEM((RB, W), jnp.float32),
            pltpu.VMEM((RB, W), jnp.float32),
            pltpu.SemaphoreType.DMA,
            pltpu.SemaphoreType.DMA,
            pltpu.SemaphoreType.DMA,
            pltpu.SemaphoreType.DMA,
        ],
        compiler_params=pltpu.CompilerParams(use_tc_tiling_on_sc=True),
    )
    def gather_slices(src_hbm, out_hbm, buf0, buf1, g0, g1, s0, s1):
        wid = lax.axis_index("s") * NC + lax.axis_index("c")
        bufs = (buf0, buf1)
        gsems = (g0, g1)
        ssems = (s0, s1)

        def task_refs(k):
            task = wid * TPW + k
            sl = task // CHUNKS
            q = task % CHUNKS
            c = sl // TS
            t = sl % TS
            t_src = (t * (T - 1)) // (TS - 1)
            rows = pl.ds(q * RB, RB)
            return (src_hbm.at[c, t_src, rows, :],
                    out_hbm.at[c, t, rows, :])

        gathers = [None, None]
        scatters = [None, None]
        for k in range(TPW):
            b = k % 2
            src_ref, dst_ref = task_refs(k)
            if scatters[b] is not None:
                scatters[b].wait()
            gathers[b] = pltpu.async_copy(src_ref, bufs[b], gsems[b])
            gathers[b].wait()
            scatters[b] = pltpu.async_copy(bufs[b], dst_ref, ssems[b])
        for b in range(2):
            if scatters[b] is not None:
                scatters[b].wait()

    slow = gather_slices(frames)
    fast = _fast_copy(frames)
    return (slow, fast)


# speedup vs baseline: 17.7768x; 17.7768x over previous
import functools

import jax
import jax.numpy as jnp
from jax import lax
from jax.experimental import pallas as pl
from jax.experimental.pallas import tpu as pltpu
from jax.experimental.pallas import tpu_sc as plsc

_ALPHA = 4


def _copy_block_body(src, dst):
    dst[...] = src[...]


def _fast_copy(frames):
    C, T, H, W = frames.shape
    TB = 4
    return pl.pallas_call(
        _copy_block_body,
        grid=(C, T // TB),
        out_shape=jax.ShapeDtypeStruct(frames.shape, frames.dtype),
        in_specs=[pl.BlockSpec((1, TB, H, W), lambda c, t: (c, t, 0, 0))],
        out_specs=pl.BlockSpec((1, TB, H, W), lambda c, t: (c, t, 0, 0)),
    )(frames)


def kernel(frames):
    C, T, H, W = frames.shape
    TS = T // _ALPHA
    NSLICES = C * TS
    CHUNKS = 4
    RB = H // CHUNKS

    info = plsc.get_sparse_core_info()
    NC, NS = info.num_cores, info.num_subcores
    NW = NC * NS
    NTASK = NSLICES * CHUNKS
    TPW = NTASK // NW

    mesh = plsc.VectorSubcoreMesh(core_axis_name="c", subcore_axis_name="s")

    @functools.partial(
        pl.kernel,
        mesh=mesh,
        out_type=jax.ShapeDtypeStruct((C, TS, H, W), jnp.float32),
        scratch_types=[
            pltpu.VMEM((RB, W), jnp.float32),
            pltpu.VMEM((RB, W), jnp.float32),
            pltpu.SemaphoreType.DMA,
            pltpu.SemaphoreType.DMA,
            pltpu.SemaphoreType.DMA,
            pltpu.SemaphoreType.DMA,
        ],
        compiler_params=pltpu.CompilerParams(use_tc_tiling_on_sc=True),
    )
    def gather_slices(src_hbm, out_hbm, buf0, buf1, g0, g1, s0, s1):
        wid = lax.axis_index("s") * NC + lax.axis_index("c")
        bufs = (buf0, buf1)
        gsems = (g0, g1)
        ssems = (s0, s1)

        def task_refs(k):
            task = wid * TPW + k
            sl = task // CHUNKS
            q = task % CHUNKS
            c = sl // TS
            t = sl % TS
            t_src = (t * (T - 1)) // (TS - 1)
            rows = pl.ds(q * RB, RB)
            return (src_hbm.at[c, t_src, rows, :],
                    out_hbm.at[c, t, rows, :])

        gathers = [None, None]
        scatters = [None, None]
        for k in range(TPW):
            b = k % 2
            src_ref, dst_ref = task_refs(k)
            if scatters[b] is not None:
                scatters[b].wait()
            gathers[b] = pltpu.async_copy(src_ref, bufs[b], gsems[b])
            gathers[b].wait()
            scatters[b] = pltpu.async_copy(bufs[b], dst_ref, ssems[b])
        for b in range(2):
            if scatters[b] is not None:
                scatters[b].wait()

    slow = gather_slices(frames)
    fast = _fast_copy(frames)
    return (slow, fast)
